# baseline (speedup 1.0000x reference)
import functools

import jax
import jax.numpy as jnp
from jax.experimental import pallas as pl
from jax.experimental.pallas import tpu as pltpu

_TINY = 1.1754943508222875e-38
_BIG_I32 = 2**31 - 1


def _threefry_bits(f_u32):
    ks0 = jnp.uint32(0)
    ks1 = jnp.uint32(42)
    ks2 = jnp.uint32(0x1BD11BDA) ^ ks0 ^ ks1
    ks = (ks0, ks1, ks2)
    rotations = ((13, 15, 26, 6), (17, 29, 16, 24))

    x0 = jnp.zeros_like(f_u32) + ks0
    x1 = f_u32 + ks1
    for i in range(5):
        for r in rotations[i % 2]:
            x0 = x0 + x1
            x1 = (x1 << jnp.uint32(r)) | (x1 >> jnp.uint32(32 - r))
            x1 = x1 ^ x0
        x0 = x0 + ks[(i + 1) % 3]
        x1 = x1 + ks[(i + 2) % 3] + jnp.uint32(i + 1)
    return x0 ^ x1


def _gumbel_from_bits(bits):
    fb = (bits >> jnp.uint32(9)) | jnp.uint32(0x3F800000)
    u01 = jax.lax.bitcast_convert_type(fb, jnp.float32) - jnp.float32(1.0)
    tiny = jnp.float32(_TINY)
    u = jnp.maximum(tiny, u01 * (jnp.float32(1.0) - tiny) + tiny)
    return -jnp.log(-jnp.log(u))


def _sample_kernel(V, R, VC, lp_ref, sc_ref, idx_ref, score_ref,
                   m_scr, i_scr, lp_scr):
    i = pl.program_id(0)
    j = pl.program_id(1)
    nj = pl.num_programs(1)

    lp = lp_ref[...]
    col = jax.lax.broadcasted_iota(jnp.int32, (R, VC), 1) + j * VC
    row = jax.lax.broadcasted_iota(jnp.int32, (R, VC), 0) + i * R
    f = row * V + col
    bits = _threefry_bits(jax.lax.bitcast_convert_type(f, jnp.uint32))
    g = _gumbel_from_bits(bits)
    val = jnp.where(col < V, g + lp, -jnp.inf)

    bm = jnp.max(val, axis=1, keepdims=True)
    is_max = val == bm
    bidx = jnp.min(jnp.where(is_max, col, jnp.int32(_BIG_I32)), axis=1, keepdims=True)
    blp = jnp.max(jnp.where(col == bidx, lp, -jnp.inf), axis=1, keepdims=True)

    @pl.when(j == 0)
    def _init():
        m_scr[...] = bm
        i_scr[...] = bidx
        lp_scr[...] = blp

    @pl.when(j > 0)
    def _update():
        better = bm > m_scr[...]
        m_scr[...] = jnp.where(better, bm, m_scr[...])
        i_scr[...] = jnp.where(better, bidx, i_scr[...])
        lp_scr[...] = jnp.where(better, blp, lp_scr[...])

    @pl.when(j == nj - 1)
    def _finalize():
        idx_ref[...] = i_scr[...].reshape(1, 1, R)
        score_ref[...] = lp_scr[...].reshape(1, 1, R) + sc_ref[...]


def kernel(step, lprobs, scores):
    bsz, beam_size, V = lprobs.shape
    NROWS = bsz * beam_size
    R = 8
    VC = 2048
    ni, nj = NROWS // R, -(-V // VC)

    lp2 = lprobs.reshape(NROWS, V)
    sc = scores.reshape(NROWS).reshape(ni, 1, R).astype(jnp.float32)

    idx3, score3 = pl.pallas_call(
        functools.partial(_sample_kernel, V, R, VC),
        grid=(ni, nj),
        in_specs=[
            pl.BlockSpec((R, VC), lambda i, j: (i, j)),
            pl.BlockSpec((1, 1, R), lambda i, j: (i, 0, 0)),
        ],
        out_specs=[
            pl.BlockSpec((1, 1, R), lambda i, j: (i, 0, 0)),
            pl.BlockSpec((1, 1, R), lambda i, j: (i, 0, 0)),
        ],
        out_shape=[
            jax.ShapeDtypeStruct((ni, 1, R), jnp.int32),
            jax.ShapeDtypeStruct((ni, 1, R), jnp.float32),
        ],
        scratch_shapes=[
            pltpu.VMEM((R, 1), jnp.float32),
            pltpu.VMEM((R, 1), jnp.int32),
            pltpu.VMEM((R, 1), jnp.float32),
        ],
        compiler_params=pltpu.CompilerParams(
            dimension_semantics=("arbitrary", "arbitrary"),
        ),
    )(lp2, sc)

    indices_buf = idx3.reshape(bsz, beam_size)
    scores_buf = score3.reshape(bsz, beam_size)
    beams_buf = jnp.tile(jnp.arange(beam_size, dtype=indices_buf.dtype), (bsz, 1))
    return (scores_buf, indices_buf, beams_buf)
